# Initial kernel scaffold; baseline (speedup 1.0000x reference)
import jax
import jax.numpy as jnp
from jax.experimental import pallas as pl


def kernel(anchor_re, anchor_im, positive_re, positive_im, negative_re, negative_im):
    raise NotImplementedError("write your pallas kernel here")


# speedup vs baseline: 1.0515x; 1.0515x over previous
import math

import jax
import jax.numpy as jnp
from jax.experimental import pallas as pl
from jax.experimental.pallas import tpu as pltpu

_N, _D = 131072, 256
_MARGIN = 1.0
_EPS = 1e-8
_BLOCK = 1024
_LANES = 128
_INV_PI = 1.0 / math.pi


def _stats_body(ar_ref, ai_ref, pr_ref, pi_ref, nr_ref, ni_ref,
                aa_ref, pp_ref, nn_ref, apre_ref, apim_ref, anre_ref,
                anim_ref):
    ar = ar_ref[...]
    ai = ai_ref[...]
    pr = pr_ref[...]
    pi = pi_ref[...]
    nr = nr_ref[...]
    ni = ni_ref[...]
    aa_ref[...] = jnp.sum(ar * ar + ai * ai, axis=1, keepdims=True)
    pp_ref[...] = jnp.sum(pr * pr + pi * pi, axis=1, keepdims=True)
    nn_ref[...] = jnp.sum(nr * nr + ni * ni, axis=1, keepdims=True)
    apre_ref[...] = jnp.sum(ar * pr + ai * pi, axis=1, keepdims=True)
    apim_ref[...] = jnp.sum(ai * pr - ar * pi, axis=1, keepdims=True)
    anre_ref[...] = jnp.sum(ar * nr + ai * ni, axis=1, keepdims=True)
    anim_ref[...] = jnp.sum(ai * nr - ar * ni, axis=1, keepdims=True)


def _acos_parts(x, y):
    wr = 1.0 - (x * x - y * y)
    wi = -2.0 * x * y
    r = jnp.sqrt(wr * wr + wi * wi)
    sr = jnp.sqrt(jnp.maximum((r + wr) * 0.5, 0.0))
    si = jnp.sign(wi) * jnp.sqrt(jnp.maximum((r - wr) * 0.5, 0.0))
    ure = x - si
    uim = y + sr
    re = jnp.arctan2(uim, ure)
    im = -0.5 * jnp.log(ure * ure + uim * uim)
    return re, im


def _loss_body(aa_ref, pp_ref, nn_ref, apre_ref, apim_ref, anre_ref,
               anim_ref, out_ref):
    aa = aa_ref[...]
    pp = pp_ref[...]
    nn = nn_ref[...]
    ma = jnp.maximum(jnp.sqrt(aa), _EPS)
    mp = jnp.maximum(jnp.sqrt(pp), _EPS)
    mn = jnp.maximum(jnp.sqrt(nn), _EPS)
    inv_ap = 1.0 / (ma * mp)
    inv_an = 1.0 / (ma * mn)
    xp = apre_ref[...] * inv_ap
    yp = apim_ref[...] * inv_ap
    xn = anre_ref[...] * inv_an
    yn = anim_ref[...] * inv_an
    p_re, p_im = _acos_parts(xp, yp)
    n_re, n_im = _acos_parts(xn, yn)
    loss_r = jnp.maximum(_MARGIN + (p_re - n_re) * _INV_PI, 0.0)
    loss_i = jnp.maximum(_MARGIN + (p_im - n_im) * _INV_PI, 0.0)
    loss = jnp.sqrt(loss_r * loss_r + loss_i * loss_i)
    total = jnp.sum(loss) * (1.0 / _N)
    out_ref[...] = jnp.full((1, _LANES), total, dtype=jnp.float32)


def kernel(anchor_re, anchor_im, positive_re, positive_im,
           negative_re, negative_im):
    grid = _N // _BLOCK
    in_spec = pl.BlockSpec((_BLOCK, _D), lambda i: (i, 0))
    col_spec = pl.BlockSpec((_BLOCK, 1), lambda i: (i, 0))
    col_shape = jax.ShapeDtypeStruct((_N, 1), jnp.float32)
    stats = pl.pallas_call(
        _stats_body,
        grid=(grid,),
        in_specs=[in_spec] * 6,
        out_specs=[col_spec] * 7,
        out_shape=[col_shape] * 7,
        compiler_params=pltpu.CompilerParams(
            dimension_semantics=("arbitrary",),
        ),
        name="triplet_stats",
    )(anchor_re, anchor_im, positive_re, positive_im,
      negative_re, negative_im)

    dense = [s.reshape(_N // _LANES, _LANES) for s in stats]
    out = pl.pallas_call(
        _loss_body,
        out_shape=jax.ShapeDtypeStruct((1, _LANES), jnp.float32),
        name="triplet_loss_epilogue",
    )(*dense)
    return out[0, 0]


# speedup vs baseline: 2.0626x; 1.9615x over previous
import math

import jax
import jax.numpy as jnp
from jax.experimental import pallas as pl
from jax.experimental.pallas import tpu as pltpu

_N, _D = 131072, 256
_MARGIN = 1.0
_EPS = 1e-8
_BLOCK = 1024
_LANES = 128
_TILES = _BLOCK // _LANES
_INV_PI = 1.0 / math.pi


def _dense_rowsum(folded):
    parts = []
    for g in range(_TILES):
        tile = folded[g * _LANES:(g + 1) * _LANES, :]
        parts.append(jnp.sum(tile.T, axis=0, keepdims=True))
    return jnp.concatenate(parts, axis=0)


def _acos_parts(x, y):
    wr = 1.0 - (x * x - y * y)
    wi = -2.0 * x * y
    r = jnp.sqrt(wr * wr + wi * wi)
    sr = jnp.sqrt(jnp.maximum((r + wr) * 0.5, 0.0))
    si = jnp.sign(wi) * jnp.sqrt(jnp.maximum((r - wr) * 0.5, 0.0))
    ure = x - si
    uim = y + sr
    re = jnp.arctan2(uim, ure)
    im = -0.5 * jnp.log(ure * ure + uim * uim)
    return re, im


def _fold(hi_lo):
    return hi_lo[:, :_LANES] + hi_lo[:, _LANES:]


def _body(ar_ref, ai_ref, pr_ref, pi_ref, nr_ref, ni_ref, out_ref):
    i = pl.program_id(0)

    @pl.when(i == 0)
    def _():
        out_ref[...] = jnp.zeros_like(out_ref)

    ar = ar_ref[...]
    ai = ai_ref[...]
    pr = pr_ref[...]
    pi = pi_ref[...]
    nr = nr_ref[...]
    ni = ni_ref[...]

    aa = _dense_rowsum(_fold(ar * ar + ai * ai))
    pp = _dense_rowsum(_fold(pr * pr + pi * pi))
    nn = _dense_rowsum(_fold(nr * nr + ni * ni))
    apre = _dense_rowsum(_fold(ar * pr + ai * pi))
    apim = _dense_rowsum(_fold(ai * pr - ar * pi))
    anre = _dense_rowsum(_fold(ar * nr + ai * ni))
    anim = _dense_rowsum(_fold(ai * nr - ar * ni))

    ma = jnp.maximum(jnp.sqrt(aa), _EPS)
    mp = jnp.maximum(jnp.sqrt(pp), _EPS)
    mn = jnp.maximum(jnp.sqrt(nn), _EPS)
    inv_ap = 1.0 / (ma * mp)
    inv_an = 1.0 / (ma * mn)
    p_re, p_im = _acos_parts(apre * inv_ap, apim * inv_ap)
    n_re, n_im = _acos_parts(anre * inv_an, anim * inv_an)
    loss_r = jnp.maximum(_MARGIN + (p_re - n_re) * _INV_PI, 0.0)
    loss_i = jnp.maximum(_MARGIN + (p_im - n_im) * _INV_PI, 0.0)
    loss = jnp.sqrt(loss_r * loss_r + loss_i * loss_i)
    block_mean = jnp.sum(loss) * (1.0 / _N)
    out_ref[...] += jnp.full((1, _LANES), block_mean, dtype=jnp.float32)


def kernel(anchor_re, anchor_im, positive_re, positive_im,
           negative_re, negative_im):
    grid = _N // _BLOCK
    in_spec = pl.BlockSpec((_BLOCK, _D), lambda i: (i, 0))
    out = pl.pallas_call(
        _body,
        grid=(grid,),
        in_specs=[in_spec] * 6,
        out_specs=pl.BlockSpec((1, _LANES), lambda i: (0, 0)),
        out_shape=jax.ShapeDtypeStruct((1, _LANES), jnp.float32),
        compiler_params=pltpu.CompilerParams(
            dimension_semantics=("arbitrary",),
        ),
        name="triplet_loss_fused",
    )(anchor_re, anchor_im, positive_re, positive_im,
      negative_re, negative_im)
    return out[0, 0]


# speedup vs baseline: 2.4074x; 1.1672x over previous
import math

import jax
import jax.numpy as jnp
from jax.experimental import pallas as pl
from jax.experimental.pallas import tpu as pltpu

_N, _D = 131072, 256
_MARGIN = 1.0
_EPS = 1e-8
_BLOCK = 2048
_LANES = 128
_TILES = _BLOCK // _LANES
_INV_PI = 1.0 / math.pi


def _dense_rowsum(folded):
    parts = []
    for g in range(_TILES):
        tile = folded[g * _LANES:(g + 1) * _LANES, :]
        parts.append(jnp.sum(tile.T, axis=0, keepdims=True))
    return jnp.concatenate(parts, axis=0)


def _acos_parts(x, y):
    wr = 1.0 - (x * x - y * y)
    wi = -2.0 * x * y
    r = jnp.sqrt(wr * wr + wi * wi)
    sr = jnp.sqrt(jnp.maximum((r + wr) * 0.5, 0.0))
    si = jnp.sign(wi) * jnp.sqrt(jnp.maximum((r - wr) * 0.5, 0.0))
    ure = x - si
    uim = y + sr
    re = jnp.arctan2(uim, ure)
    im = -0.5 * jnp.log(ure * ure + uim * uim)
    return re, im


def _fold(hi_lo):
    return hi_lo[:, :_LANES] + hi_lo[:, _LANES:]


def _body(ar_ref, ai_ref, pr_ref, pi_ref, nr_ref, ni_ref, out_ref):
    i = pl.program_id(0)

    @pl.when(i == 0)
    def _():
        out_ref[...] = jnp.zeros_like(out_ref)

    ar = ar_ref[...]
    ai = ai_ref[...]
    pr = pr_ref[...]
    pi = pi_ref[...]
    nr = nr_ref[...]
    ni = ni_ref[...]

    aa = _dense_rowsum(_fold(ar * ar + ai * ai))
    pp = _dense_rowsum(_fold(pr * pr + pi * pi))
    nn = _dense_rowsum(_fold(nr * nr + ni * ni))
    apre = _dense_rowsum(_fold(ar * pr + ai * pi))
    apim = _dense_rowsum(_fold(ai * pr - ar * pi))
    anre = _dense_rowsum(_fold(ar * nr + ai * ni))
    anim = _dense_rowsum(_fold(ai * nr - ar * ni))

    ma = jnp.maximum(jnp.sqrt(aa), _EPS)
    mp = jnp.maximum(jnp.sqrt(pp), _EPS)
    mn = jnp.maximum(jnp.sqrt(nn), _EPS)
    inv_ap = 1.0 / (ma * mp)
    inv_an = 1.0 / (ma * mn)
    p_re, p_im = _acos_parts(apre * inv_ap, apim * inv_ap)
    n_re, n_im = _acos_parts(anre * inv_an, anim * inv_an)
    loss_r = jnp.maximum(_MARGIN + (p_re - n_re) * _INV_PI, 0.0)
    loss_i = jnp.maximum(_MARGIN + (p_im - n_im) * _INV_PI, 0.0)
    loss = jnp.sqrt(loss_r * loss_r + loss_i * loss_i)
    block_mean = jnp.sum(loss) * (1.0 / _N)
    out_ref[...] += jnp.full((1, _LANES), block_mean, dtype=jnp.float32)


def kernel(anchor_re, anchor_im, positive_re, positive_im,
           negative_re, negative_im):
    grid = _N // _BLOCK
    in_spec = pl.BlockSpec((_BLOCK, _D), lambda i: (i, 0))
    out = pl.pallas_call(
        _body,
        grid=(grid,),
        in_specs=[in_spec] * 6,
        out_specs=pl.BlockSpec((1, _LANES), lambda i: (0, 0)),
        out_shape=jax.ShapeDtypeStruct((1, _LANES), jnp.float32),
        compiler_params=pltpu.CompilerParams(
            dimension_semantics=("arbitrary",),
        ),
        name="triplet_loss_fused",
    )(anchor_re, anchor_im, positive_re, positive_im,
      negative_re, negative_im)
    return out[0, 0]


# speedup vs baseline: 2.5302x; 1.0510x over previous
import math

import jax
import jax.numpy as jnp
from jax.experimental import pallas as pl
from jax.experimental.pallas import tpu as pltpu

_N, _D = 131072, 256
_MARGIN = 1.0
_EPS = 1e-8
_BLOCK = 4096
_LANES = 128
_TILES = _BLOCK // _LANES
_INV_PI = 1.0 / math.pi


def _dense_rowsum(folded):
    parts = []
    for g in range(_TILES):
        tile = folded[g * _LANES:(g + 1) * _LANES, :]
        parts.append(jnp.sum(tile.T, axis=0, keepdims=True))
    return jnp.concatenate(parts, axis=0)


def _acos_parts(x, y):
    wr = 1.0 - (x * x - y * y)
    wi = -2.0 * x * y
    r = jnp.sqrt(wr * wr + wi * wi)
    sr = jnp.sqrt(jnp.maximum((r + wr) * 0.5, 0.0))
    si = jnp.sign(wi) * jnp.sqrt(jnp.maximum((r - wr) * 0.5, 0.0))
    ure = x - si
    uim = y + sr
    re = jnp.arctan2(uim, ure)
    im = -0.5 * jnp.log(ure * ure + uim * uim)
    return re, im


def _fold(hi_lo):
    return hi_lo[:, :_LANES] + hi_lo[:, _LANES:]


def _body(ar_ref, ai_ref, pr_ref, pi_ref, nr_ref, ni_ref, out_ref):
    i = pl.program_id(0)

    @pl.when(i == 0)
    def _():
        out_ref[...] = jnp.zeros_like(out_ref)

    ar = ar_ref[...]
    ai = ai_ref[...]
    pr = pr_ref[...]
    pi = pi_ref[...]
    nr = nr_ref[...]
    ni = ni_ref[...]

    aa = _dense_rowsum(_fold(ar * ar + ai * ai))
    pp = _dense_rowsum(_fold(pr * pr + pi * pi))
    nn = _dense_rowsum(_fold(nr * nr + ni * ni))
    apre = _dense_rowsum(_fold(ar * pr + ai * pi))
    apim = _dense_rowsum(_fold(ai * pr - ar * pi))
    anre = _dense_rowsum(_fold(ar * nr + ai * ni))
    anim = _dense_rowsum(_fold(ai * nr - ar * ni))

    ma = jnp.maximum(jnp.sqrt(aa), _EPS)
    mp = jnp.maximum(jnp.sqrt(pp), _EPS)
    mn = jnp.maximum(jnp.sqrt(nn), _EPS)
    inv_ap = 1.0 / (ma * mp)
    inv_an = 1.0 / (ma * mn)
    p_re, p_im = _acos_parts(apre * inv_ap, apim * inv_ap)
    n_re, n_im = _acos_parts(anre * inv_an, anim * inv_an)
    loss_r = jnp.maximum(_MARGIN + (p_re - n_re) * _INV_PI, 0.0)
    loss_i = jnp.maximum(_MARGIN + (p_im - n_im) * _INV_PI, 0.0)
    loss = jnp.sqrt(loss_r * loss_r + loss_i * loss_i)
    block_mean = jnp.sum(loss) * (1.0 / _N)
    out_ref[...] += jnp.full((1, _LANES), block_mean, dtype=jnp.float32)


def kernel(anchor_re, anchor_im, positive_re, positive_im,
           negative_re, negative_im):
    grid = _N // _BLOCK
    in_spec = pl.BlockSpec((_BLOCK, _D), lambda i: (i, 0))
    out = pl.pallas_call(
        _body,
        grid=(grid,),
        in_specs=[in_spec] * 6,
        out_specs=pl.BlockSpec((1, _LANES), lambda i: (0, 0)),
        out_shape=jax.ShapeDtypeStruct((1, _LANES), jnp.float32),
        compiler_params=pltpu.CompilerParams(
            dimension_semantics=("arbitrary",),
        ),
        name="triplet_loss_fused",
    )(anchor_re, anchor_im, positive_re, positive_im,
      negative_re, negative_im)
    return out[0, 0]
